# Initial kernel scaffold; baseline (speedup 1.0000x reference)
import jax
import jax.numpy as jnp
from jax import lax
from jax.experimental import pallas as pl
from jax.experimental.pallas import tpu as pltpu
from jax.experimental.pallas import tpu_sc as plsc

M = 100000
B = 16384
D = 128

HALF = M // 2
P = 12800
NPASS = 4
CH = 128
TROWS = P // 16
NVREG = B // 16
LANES = 16


def _scatter_add_kernel(var_hbm, value_hbm, idx_hbm, alpha_hbm, out_hbm,
                        acc_sh, idx_v, vchunk, locidx, alpha_v):
    c = lax.axis_index("c")
    s = lax.axis_index("s")

    pltpu.sync_copy(idx_hbm, idx_v)
    pltpu.sync_copy(alpha_hbm, alpha_v)
    avec = alpha_v[...]
    alpha_s = lax.reduce_max(avec, axes=(0,))

    lane_iota = lax.iota(jnp.int32, LANES)

    def lower_bound(bound):
        def cond(carry):
            lo, hi = carry
            return lo < hi

        def body(carry):
            lo, hi = carry
            mid = lax.div(lo + hi, jnp.int32(2))
            v = idx_v[pl.ds(mid * LANES, LANES)]
            cnt = jnp.sum((v < bound).astype(jnp.int32))
            all_lt = cnt == LANES
            return (jnp.where(all_lt, mid + 1, lo),
                    jnp.where(all_lt, hi, mid))

        lo, _ = lax.while_loop(cond, body,
                               (jnp.int32(0), jnp.int32(NVREG)))
        safe = jnp.minimum(lo, jnp.int32(NVREG - 1))
        v = idx_v[pl.ds(safe * LANES, LANES)]
        cnt = jnp.sum((v < bound).astype(jnp.int32))
        return jnp.where(lo >= NVREG, jnp.int32(B), lo * LANES + cnt)

    row_base = c * HALF
    rstart = s * TROWS

    for p in range(NPASS):
        woff = min(p * P, HALF - P)
        w = row_base + woff

        pltpu.sync_copy(var_hbm.at[pl.ds(w + rstart, TROWS), :],
                        acc_sh.at[pl.ds(rstart, TROWS), :])
        plsc.subcore_barrier()

        jlo = lower_bound(w)
        jhi = lower_bound(w + P)
        seg = jhi - jlo
        share = lax.div(seg + jnp.int32(16 * CH - 1),
                        jnp.int32(16 * CH)) * CH
        a = jlo + s * share
        b = jnp.minimum(a + share, jhi)
        kk0 = lax.div(a, jnp.int32(CH)) * CH

        def chunk_body(kk):
            pltpu.sync_copy(value_hbm.at[pl.ds(kk, CH), :], vchunk)

            @pl.when(alpha_s != 1.0)
            def _():
                @pl.loop(0, CH)
                def _(r):
                    for m in range(D // LANES):
                        sl = pl.ds(m * LANES, LANES)
                        vchunk[r, sl] = vchunk[r, sl] * avec

            for m in range(CH // LANES):
                iv = idx_v[pl.ds(kk + m * LANES, LANES)]
                jpos = kk + m * LANES + lane_iota
                valid = (jpos >= a) & (jpos < b)
                dump = jnp.int32(P) + m * LANES + lane_iota
                locidx[pl.ds(m * LANES, LANES)] = (
                    jnp.where(valid, iv - w, dump))
            pltpu.sync_copy(vchunk, acc_sh.at[locidx], add=True)
            return kk + CH

        lax.while_loop(lambda kk: kk < b, chunk_body, kk0)
        plsc.subcore_barrier()

        pltpu.sync_copy(acc_sh.at[pl.ds(rstart, TROWS), :],
                        out_hbm.at[pl.ds(w + rstart, TROWS), :])
        plsc.subcore_barrier()


def kernel(var, value, sorted_indices, pos, alpha):
    del pos
    alpha_vec = jnp.broadcast_to(
        jnp.asarray(alpha, jnp.float32).reshape(1), (LANES,))

    mesh = plsc.VectorSubcoreMesh(core_axis_name="c", subcore_axis_name="s")
    run = pl.kernel(
        _scatter_add_kernel,
        out_type=jax.ShapeDtypeStruct((M, D), jnp.float32),
        mesh=mesh,
        scratch_types=[
            pltpu.VMEM_SHARED((P + CH, D), jnp.float32),
            pltpu.VMEM((B,), jnp.int32),
            pltpu.VMEM((CH, D), jnp.float32),
            pltpu.VMEM((CH,), jnp.int32),
            pltpu.VMEM((LANES,), jnp.float32),
        ],
    )
    return run(var, value, sorted_indices.astype(jnp.int32), alpha_vec)


# speedup vs baseline: 1.4314x; 1.4314x over previous
import dataclasses

import jax
import jax.numpy as jnp
from jax import lax
from jax.experimental import pallas as pl
from jax.experimental.pallas import tpu as pltpu
from jax.experimental.pallas import tpu_sc as plsc

M = 100000
B = 16384
D = 128

HALF = M // 2
P = 10240
NPASS = 5
CH = 128
TROWS = P // 16
NVREG = B // 16
LANES = 16


def _scatter_add_kernel(var_hbm, value_hbm, idx_hbm, alpha_hbm, out_hbm,
                        acc_sh, idx_v, vchunk, locidx, alpha_v):
    c = lax.axis_index("c")
    s = lax.axis_index("s")

    pltpu.sync_copy(idx_hbm, idx_v)
    pltpu.sync_copy(alpha_hbm, alpha_v)
    avec = alpha_v[...]
    alpha_s = lax.reduce_max(avec, axes=(0,))

    lane_iota = lax.iota(jnp.int32, LANES)

    def lower_bound(bound):
        def cond(carry):
            lo, hi = carry
            return lo < hi

        def body(carry):
            lo, hi = carry
            mid = lax.div(lo + hi, jnp.int32(2))
            v = idx_v[pl.ds(mid * LANES, LANES)]
            cnt = jnp.sum((v < bound).astype(jnp.int32))
            all_lt = cnt == LANES
            return (jnp.where(all_lt, mid + 1, lo),
                    jnp.where(all_lt, hi, mid))

        lo, _ = lax.while_loop(cond, body,
                               (jnp.int32(0), jnp.int32(NVREG)))
        safe = jnp.minimum(lo, jnp.int32(NVREG - 1))
        v = idx_v[pl.ds(safe * LANES, LANES)]
        cnt = jnp.sum((v < bound).astype(jnp.int32))
        return jnp.where(lo >= NVREG, jnp.int32(B), lo * LANES + cnt)

    row_base = pl.multiple_of(c * HALF, 8)
    rstart = pl.multiple_of(s * TROWS, 8)

    for p in range(NPASS):
        woff = min(p * P, HALF - P)
        w = pl.multiple_of(row_base + woff, 8)

        pltpu.sync_copy(var_hbm.at[pl.ds(w + rstart, TROWS), :],
                        acc_sh.at[pl.ds(rstart, TROWS), :])
        plsc.subcore_barrier()

        jlo = lower_bound(w)
        jhi = lower_bound(w + P)
        seg = jhi - jlo
        share = lax.div(seg + jnp.int32(16 * CH - 1),
                        jnp.int32(16 * CH)) * CH
        a = jlo + s * share
        b = jnp.minimum(a + share, jhi)
        kk0 = lax.div(a, jnp.int32(CH)) * CH

        def chunk_body(kk):
            kk = pl.multiple_of(kk, CH)
            pltpu.sync_copy(value_hbm.at[pl.ds(kk, CH), :], vchunk)

            @pl.when(alpha_s != 1.0)
            def _():
                @pl.loop(0, CH)
                def _(r):
                    for m in range(D // LANES):
                        sl = pl.ds(m * LANES, LANES)
                        vchunk[r, sl] = vchunk[r, sl] * avec

            for m in range(CH // LANES):
                iv = idx_v[pl.ds(kk + m * LANES, LANES)]
                jpos = kk + m * LANES + lane_iota
                valid = (jpos >= a) & (jpos < b)
                dump = jnp.int32(P) + m * LANES + lane_iota
                locidx[pl.ds(m * LANES, LANES)] = (
                    jnp.where(valid, iv - w, dump))
            pltpu.sync_copy(vchunk, acc_sh.at[locidx], add=True)
            return kk + CH

        lax.while_loop(lambda kk: kk < b, chunk_body, kk0)
        plsc.subcore_barrier()

        pltpu.sync_copy(acc_sh.at[pl.ds(rstart, TROWS), :],
                        out_hbm.at[pl.ds(w + rstart, TROWS), :])
        plsc.subcore_barrier()


def kernel(var, value, sorted_indices, pos, alpha):
    del pos
    alpha_vec = jnp.broadcast_to(
        jnp.asarray(alpha, jnp.float32).reshape(1), (LANES,))

    cp = pltpu.CompilerParams()
    if "needs_layout_passes" in pltpu.CompilerParams.__dataclass_fields__:
        cp = dataclasses.replace(cp, needs_layout_passes=False)

    mesh = plsc.VectorSubcoreMesh(core_axis_name="c", subcore_axis_name="s")
    run = pl.kernel(
        _scatter_add_kernel,
        out_type=jax.ShapeDtypeStruct((M, D), jnp.float32),
        mesh=mesh,
        scratch_types=[
            pltpu.VMEM_SHARED((P + CH, D), jnp.float32),
            pltpu.VMEM((B,), jnp.int32),
            pltpu.VMEM((CH, D), jnp.float32),
            pltpu.VMEM((CH,), jnp.int32),
            pltpu.VMEM((LANES,), jnp.float32),
        ],
        compiler_params=cp,
    )
    return run(var, value, sorted_indices.astype(jnp.int32), alpha_vec)


# speedup vs baseline: 1.6083x; 1.1236x over previous
import dataclasses

import jax
import jax.numpy as jnp
from jax import lax
from jax.experimental import pallas as pl
from jax.experimental.pallas import tpu as pltpu
from jax.experimental.pallas import tpu_sc as plsc

M = 100000
B = 16384
D = 128

HALF = M // 2
P = 5888
NPASS = 9
CH = 128
TROWS = P // 16
NVREG = B // 16
LANES = 16


def _scatter_add_kernel(var_hbm, value_hbm, idx_hbm, alpha_hbm, out_hbm,
                        acc0, acc1, idx_v, vchunk, locidx, alpha_v, sems):
    c = lax.axis_index("c")
    s = lax.axis_index("s")

    pltpu.sync_copy(idx_hbm, idx_v)
    pltpu.sync_copy(alpha_hbm, alpha_v)
    avec = alpha_v[...]
    alpha_s = lax.reduce_max(avec, axes=(0,))

    lane_iota = lax.iota(jnp.int32, LANES)

    def lower_bound(bound):
        def cond(carry):
            lo, hi = carry
            return lo < hi

        def body(carry):
            lo, hi = carry
            mid = lax.div(lo + hi, jnp.int32(2))
            v = idx_v[pl.ds(mid * LANES, LANES)]
            cnt = jnp.sum((v < bound).astype(jnp.int32))
            all_lt = cnt == LANES
            return (jnp.where(all_lt, mid + 1, lo),
                    jnp.where(all_lt, hi, mid))

        lo, _ = lax.while_loop(cond, body,
                               (jnp.int32(0), jnp.int32(NVREG)))
        safe = jnp.minimum(lo, jnp.int32(NVREG - 1))
        v = idx_v[pl.ds(safe * LANES, LANES)]
        cnt = jnp.sum((v < bound).astype(jnp.int32))
        return jnp.where(lo >= NVREG, jnp.int32(B), lo * LANES + cnt)

    row_base = pl.multiple_of(c * HALF, 8)
    rstart = pl.multiple_of(s * TROWS, 8)
    accs = (acc0, acc1)
    wb_descs = [None] * NPASS

    for p in range(NPASS):
        buf = p % 2
        acc_sh = accs[buf]
        woff = min(p * P, HALF - P)
        w = pl.multiple_of(row_base + woff, 8)

        if p >= 2:
            wb_descs[p - 2].wait()
        init_desc = pltpu.async_copy(
            var_hbm.at[pl.ds(w + rstart, TROWS), :],
            acc_sh.at[pl.ds(rstart, TROWS), :],
            sems.at[buf])

        jlo = lower_bound(w)
        jhi = lower_bound(w + P)
        seg = jhi - jlo
        share = lax.div(seg + jnp.int32(16 * CH - 1),
                        jnp.int32(16 * CH)) * CH
        a = jlo + s * share
        b = jnp.minimum(a + share, jhi)
        kk0 = lax.div(a, jnp.int32(CH)) * CH

        init_desc.wait()
        plsc.subcore_barrier()

        def chunk_body(kk):
            kk = pl.multiple_of(kk, CH)
            pltpu.sync_copy(value_hbm.at[pl.ds(kk, CH), :], vchunk)

            @pl.when(alpha_s != 1.0)
            def _():
                @pl.loop(0, CH)
                def _(r):
                    for m in range(D // LANES):
                        sl = pl.ds(m * LANES, LANES)
                        vchunk[r, sl] = vchunk[r, sl] * avec

            for m in range(CH // LANES):
                iv = idx_v[pl.ds(kk + m * LANES, LANES)]
                jpos = kk + m * LANES + lane_iota
                valid = (jpos >= a) & (jpos < b)
                dump = jnp.int32(P) + m * LANES + lane_iota
                locidx[pl.ds(m * LANES, LANES)] = (
                    jnp.where(valid, iv - w, dump))
            pltpu.sync_copy(vchunk, acc_sh.at[locidx], add=True)
            return kk + CH

        lax.while_loop(lambda kk: kk < b, chunk_body, kk0)
        plsc.subcore_barrier()

        wb_descs[p] = pltpu.async_copy(
            acc_sh.at[pl.ds(rstart, TROWS), :],
            out_hbm.at[pl.ds(w + rstart, TROWS), :],
            sems.at[2 + buf])

    wb_descs[NPASS - 2].wait()
    wb_descs[NPASS - 1].wait()


def kernel(var, value, sorted_indices, pos, alpha):
    del pos
    alpha_vec = jnp.broadcast_to(
        jnp.asarray(alpha, jnp.float32).reshape(1), (LANES,))

    cp = pltpu.CompilerParams()
    if "needs_layout_passes" in pltpu.CompilerParams.__dataclass_fields__:
        cp = dataclasses.replace(cp, needs_layout_passes=False)

    mesh = plsc.VectorSubcoreMesh(core_axis_name="c", subcore_axis_name="s")
    run = pl.kernel(
        _scatter_add_kernel,
        out_type=jax.ShapeDtypeStruct((M, D), jnp.float32),
        mesh=mesh,
        scratch_types=[
            pltpu.VMEM_SHARED((P + CH, D), jnp.float32),
            pltpu.VMEM_SHARED((P + CH, D), jnp.float32),
            pltpu.VMEM((B,), jnp.int32),
            pltpu.VMEM((CH, D), jnp.float32),
            pltpu.VMEM((CH,), jnp.int32),
            pltpu.VMEM((LANES,), jnp.float32),
            pltpu.SemaphoreType.DMA((4,)),
        ],
        compiler_params=cp,
    )
    return run(var, value, sorted_indices.astype(jnp.int32), alpha_vec)


# speedup vs baseline: 1.7449x; 1.0849x over previous
import dataclasses

import jax
import jax.numpy as jnp
from jax import lax
from jax.experimental import pallas as pl
from jax.experimental.pallas import tpu as pltpu
from jax.experimental.pallas import tpu_sc as plsc

M = 100000
B = 16384
D = 128

NTILES = 32
RPT = 3200
W = 320
NW = RPT // W
CH = 128
NVREG = B // 16
LANES = 16

_GATHER_DNUMS = lax.GatherDimensionNumbers(
    offset_dims=(), collapsed_slice_dims=(0,), start_index_map=(0,))


def _bcast16(vals, pos):
    idx = jnp.full((LANES,), pos, jnp.int32)
    return lax.gather(vals, idx[:, None], _GATHER_DNUMS, slice_sizes=(1,),
                      mode=lax.GatherScatterMode.PROMISE_IN_BOUNDS)


def _scatter_add_kernel(var_hbm, value_hbm, idx_hbm, alpha_hbm, out_hbm,
                        buf0, buf1, idx_v, vchunk, alpha_v, sems):
    c = lax.axis_index("c")
    s = lax.axis_index("s")
    tg = s * 2 + c

    pltpu.sync_copy(idx_hbm, idx_v)
    pltpu.sync_copy(alpha_hbm, alpha_v)
    avec = alpha_v[...]
    alpha_s = lax.reduce_max(avec, axes=(0,))

    lane_iota = lax.iota(jnp.int32, LANES)
    start = pl.multiple_of(
        jnp.minimum(tg * RPT, jnp.int32(M - RPT)), 8)

    def lower_bound(bound):
        def cond(carry):
            lo, hi = carry
            return lo < hi

        def body(carry):
            lo, hi = carry
            mid = lax.div(lo + hi, jnp.int32(2))
            v = idx_v[pl.ds(mid * LANES, LANES)]
            cnt = jnp.sum((v < bound).astype(jnp.int32))
            all_lt = cnt == LANES
            return (jnp.where(all_lt, mid + 1, lo),
                    jnp.where(all_lt, hi, mid))

        lo, _ = lax.while_loop(cond, body,
                               (jnp.int32(0), jnp.int32(NVREG)))
        safe = jnp.minimum(lo, jnp.int32(NVREG - 1))
        v = idx_v[pl.ds(safe * LANES, LANES)]
        cnt = jnp.sum((v < bound).astype(jnp.int32))
        return jnp.where(lo >= NVREG, jnp.int32(B), lo * LANES + cnt)

    bufs = (buf0, buf1)

    jb = [lower_bound(start + k * W) for k in range(NW + 1)]

    def win_rows(k):
        return pl.multiple_of(start + k * W, 8)

    init_descs = [None] * NW
    wb_descs = [None] * NW
    init_descs[0] = pltpu.async_copy(
        var_hbm.at[pl.ds(win_rows(0), W), :], bufs[0], sems.at[0])

    for k in range(NW):
        b = k % 2
        buf = bufs[b]
        init_descs[k].wait()
        if k + 1 < NW:
            if k >= 1:
                wb_descs[k - 1].wait()
            init_descs[k + 1] = pltpu.async_copy(
                var_hbm.at[pl.ds(win_rows(k + 1), W), :],
                bufs[(k + 1) % 2], sems.at[(k + 1) % 2])

        w = win_rows(k)
        jlo = jb[k]
        jhi = jb[k + 1]
        kk0 = lax.div(jlo, jnp.int32(CH)) * CH

        def chunk_body(kk):
            kk = pl.multiple_of(kk, CH)
            pltpu.sync_copy(value_hbm.at[pl.ds(kk, CH), :], vchunk)

            @pl.when(alpha_s != 1.0)
            def _():
                @pl.loop(0, CH)
                def _(r):
                    for m in range(D // LANES):
                        sl = pl.ds(m * LANES, LANES)
                        vchunk[r, sl] = vchunk[r, sl] * avec

            lo_c = jnp.maximum(jlo, kk)
            hi_c = jnp.minimum(jhi, kk + CH)

            def j_body(j, _):
                jr = j - kk
                grp = lax.div(jr, jnp.int32(LANES)) * LANES
                iv = idx_v[pl.ds(kk + grp, LANES)]
                rowv = _bcast16(iv, jr - grp) - w
                for m in range(D // LANES):
                    x = vchunk[jr, pl.ds(m * LANES, LANES)]
                    plsc.addupdate_scatter(
                        buf, [rowv, m * LANES + lane_iota], x)
                return 0

            lax.fori_loop(lo_c, hi_c, j_body, 0)
            return kk + CH

        lax.while_loop(lambda kk: kk < jhi, chunk_body, kk0)

        wb_descs[k] = pltpu.async_copy(
            buf, out_hbm.at[pl.ds(w, W), :], sems.at[2 + b])

    wb_descs[NW - 2].wait()
    wb_descs[NW - 1].wait()


def kernel(var, value, sorted_indices, pos, alpha):
    del pos
    alpha_vec = jnp.broadcast_to(
        jnp.asarray(alpha, jnp.float32).reshape(1), (LANES,))

    cp = pltpu.CompilerParams()
    if "needs_layout_passes" in pltpu.CompilerParams.__dataclass_fields__:
        cp = dataclasses.replace(cp, needs_layout_passes=False)

    mesh = plsc.VectorSubcoreMesh(core_axis_name="c", subcore_axis_name="s")
    run = pl.kernel(
        _scatter_add_kernel,
        out_type=jax.ShapeDtypeStruct((M, D), jnp.float32),
        mesh=mesh,
        scratch_types=[
            pltpu.VMEM((W, D), jnp.float32),
            pltpu.VMEM((W, D), jnp.float32),
            pltpu.VMEM((B,), jnp.int32),
            pltpu.VMEM((CH, D), jnp.float32),
            pltpu.VMEM((LANES,), jnp.float32),
            pltpu.SemaphoreType.DMA((4,)),
        ],
        compiler_params=cp,
    )
    return run(var, value, sorted_indices.astype(jnp.int32), alpha_vec)


# speedup vs baseline: 2.1747x; 1.2463x over previous
import dataclasses

import jax
import jax.numpy as jnp
from jax import lax
from jax.experimental import pallas as pl
from jax.experimental.pallas import tpu as pltpu
from jax.experimental.pallas import tpu_sc as plsc

M = 100000
B = 16384
D = 128

RPT = 3200
W = 160
NW = RPT // W
NBUF = 4
CH = 64
LANES = 16

_GATHER_DNUMS = lax.GatherDimensionNumbers(
    offset_dims=(), collapsed_slice_dims=(0,), start_index_map=(0,))


def _bcast16(vals, pos):
    idx = jnp.full((LANES,), pos, jnp.int32)
    return lax.gather(vals, idx[:, None], _GATHER_DNUMS, slice_sizes=(1,),
                      mode=lax.GatherScatterMode.PROMISE_IN_BOUNDS)


def _scatter_add_kernel(var_hbm, value_hbm, idx_hbm, alpha_hbm, out_hbm,
                        b0, b1, b2, b3, idx_v, vc0, vc1, alpha_v,
                        isems, wsems, vsems):
    c = lax.axis_index("c")
    s = lax.axis_index("s")
    tg = s * 2 + c

    pltpu.sync_copy(idx_hbm, idx_v)
    pltpu.sync_copy(alpha_hbm, alpha_v)
    avec = alpha_v[...]
    alpha_s = lax.reduce_max(avec, axes=(0,))

    lane_iota = lax.iota(jnp.int32, LANES)
    start = pl.multiple_of(
        jnp.minimum(tg * RPT, jnp.int32(M - RPT)), 8)

    def search16(bounds):
        lo = jnp.zeros((LANES,), jnp.int32)
        hi = jnp.full((LANES,), B, jnp.int32)
        for _ in range(15):
            live = lo < hi
            mid = lax.div(lo + hi, jnp.int32(2))
            vals = plsc.load_gather(
                idx_v, [jnp.minimum(mid, jnp.int32(B - 1))])
            pred = live & (vals < bounds)
            lo = jnp.where(pred, mid + 1, lo)
            hi = jnp.where(live & jnp.logical_not(vals < bounds), mid, hi)
        return lo

    q0 = search16(start + lane_iota * W)
    q1 = search16(start + (LANES + lane_iota) * W)

    def extract(k):
        e0 = lax.reduce_max(
            jnp.where(lane_iota == k, q0, jnp.int32(0)), axes=(0,))
        e1 = lax.reduce_max(
            jnp.where(lane_iota == k - LANES, q1, jnp.int32(0)), axes=(0,))
        return jnp.where(k < LANES, e0, e1)

    bufs = (b0, b1, b2, b3)
    vcs = (vc0, vc1)
    cols = [m * LANES + lane_iota for m in range(D // LANES)]

    def win_rows(k):
        return pl.multiple_of(start + k * W, 8)

    def chunk_start(j):
        return pl.multiple_of(
            jnp.minimum(lax.div(j, jnp.int32(CH)) * CH, jnp.int32(B - CH)), 8)

    pltpu.async_copy(var_hbm.at[pl.ds(win_rows(0), W), :], bufs[0],
                     isems.at[0])
    pltpu.async_copy(var_hbm.at[pl.ds(win_rows(1), W), :], bufs[1],
                     isems.at[1])
    jb0 = extract(jnp.int32(0))
    pltpu.async_copy(value_hbm.at[pl.ds(chunk_start(jb0), CH), :],
                     vcs[0], vsems.at[0])

    def process_chunk(buf, vc, kk, w, jlo, jhi):
        @pl.when(alpha_s != 1.0)
        def _():
            @pl.loop(0, CH)
            def _(r):
                for m in range(D // LANES):
                    sl = pl.ds(m * LANES, LANES)
                    vc[r, sl] = vc[r, sl] * avec

        lo_c = jnp.maximum(jlo, kk)
        hi_c = jnp.minimum(jhi, kk + CH)

        def j_body(j, _):
            jr = j - kk
            grp = lax.div(jr, jnp.int32(LANES)) * LANES
            iv = idx_v[pl.ds(kk + grp, LANES)]
            rowv = _bcast16(iv, jr - grp) - w
            mask = (rowv >= 0) & (rowv < W)
            rowc = jnp.clip(rowv, 0, W - 1)
            for m in range(D // LANES):
                x = vc[jr, pl.ds(m * LANES, LANES)]
                plsc.addupdate_scatter(buf, [rowc, cols[m]], x, mask=mask)
            return 0

        lax.fori_loop(lo_c, hi_c, j_body, 0)

    @pl.loop(0, NW // NBUF)
    def _(g):
        for bslot in range(NBUF):
            k = g * NBUF + bslot
            buf = bufs[bslot]
            vslot = bslot % 2
            vc = vcs[vslot]
            w = win_rows(k)

            @pl.when(k + 2 < NW)
            def _():
                nslot = (bslot + 2) % NBUF

                @pl.when(k >= 2)
                def _():
                    pltpu.make_async_copy(
                        bufs[nslot],
                        out_hbm.at[pl.ds(win_rows(k - 2), W), :],
                        wsems.at[nslot]).wait()

                pltpu.async_copy(
                    var_hbm.at[pl.ds(win_rows(k + 2), W), :],
                    bufs[nslot], isems.at[nslot])

            jlo = extract(k)
            jhi = extract(k + 1)
            kk0 = chunk_start(jlo)

            @pl.when(k + 1 < NW)
            def _():
                pltpu.async_copy(
                    value_hbm.at[pl.ds(chunk_start(jhi), CH), :],
                    vcs[(vslot + 1) % 2], vsems.at[(vslot + 1) % 2])

            pltpu.make_async_copy(
                var_hbm.at[pl.ds(w, W), :], buf, isems.at[bslot]).wait()
            pltpu.make_async_copy(
                value_hbm.at[pl.ds(kk0, CH), :], vc, vsems.at[vslot]).wait()

            @pl.when(jlo < jhi)
            def _():
                process_chunk(buf, vc, kk0, w, jlo, jhi)

                def extra_body(kk):
                    kk = pl.multiple_of(kk, 8)
                    pltpu.sync_copy(value_hbm.at[pl.ds(kk, CH), :], vc)
                    process_chunk(buf, vc, kk, w, jlo, jhi)
                    return kk + CH

                lax.while_loop(lambda kk: kk < jhi, extra_body, kk0 + CH)

            pltpu.async_copy(buf, out_hbm.at[pl.ds(w, W), :],
                             wsems.at[bslot])

    for k in range(NW - NBUF, NW):
        pltpu.make_async_copy(
            bufs[k % NBUF],
            out_hbm.at[pl.ds(win_rows(k), W), :],
            wsems.at[k % NBUF]).wait()


def kernel(var, value, sorted_indices, pos, alpha):
    del pos
    alpha_vec = jnp.broadcast_to(
        jnp.asarray(alpha, jnp.float32).reshape(1), (LANES,))

    cp = pltpu.CompilerParams()
    if "needs_layout_passes" in pltpu.CompilerParams.__dataclass_fields__:
        cp = dataclasses.replace(cp, needs_layout_passes=False)

    mesh = plsc.VectorSubcoreMesh(core_axis_name="c", subcore_axis_name="s")
    run = pl.kernel(
        _scatter_add_kernel,
        out_type=jax.ShapeDtypeStruct((M, D), jnp.float32),
        mesh=mesh,
        scratch_types=[
            pltpu.VMEM((W, D), jnp.float32),
            pltpu.VMEM((W, D), jnp.float32),
            pltpu.VMEM((W, D), jnp.float32),
            pltpu.VMEM((W, D), jnp.float32),
            pltpu.VMEM((B,), jnp.int32),
            pltpu.VMEM((CH, D), jnp.float32),
            pltpu.VMEM((CH, D), jnp.float32),
            pltpu.VMEM((LANES,), jnp.float32),
            pltpu.SemaphoreType.DMA((NBUF,)),
            pltpu.SemaphoreType.DMA((NBUF,)),
            pltpu.SemaphoreType.DMA((2,)),
        ],
        compiler_params=cp,
    )
    return run(var, value, sorted_indices.astype(jnp.int32), alpha_vec)
